# baseline (speedup 1.0000x reference)
import math

import jax
import jax.numpy as jnp
from jax.experimental import pallas as pl
from jax.experimental.pallas import tpu as pltpu

NUM_FEATURES = 128
NUM_PIDS = 100000
NUM_CQ = 5000
OIM_SCALAR = 30.0
BATCH = 128
NUM_STEPS = 5
BLK = NUM_PIDS // NUM_STEPS
CBLK = NUM_CQ // NUM_STEPS
IGNORE_INDEX = 5554
LOG2E = math.log2(math.e)
LN2 = math.log(2.0)


def _oim_kernel(x_ref, lab_ref, lut_ref, cq_ref, relc_lut_ref, relc_cq_ref,
                out_ref, s_ref, lsc_ref):
    i = pl.program_id(0)
    x = x_ref[...]
    labels = lab_ref[...]

    def scores2(w, relc):
        lg = jax.lax.dot_general(
            x, w, (((1,), (1,)), ((), ())),
            preferred_element_type=jnp.float32,
            precision=jax.lax.Precision.DEFAULT)
        return lg * relc

    @pl.when(i == 0)
    def _init():
        s_ref[...] = jnp.zeros_like(s_ref)
        lsc_ref[...] = jnp.zeros_like(lsc_ref)

    y = scores2(lut_ref[...], relc_lut_ref[0])
    yc = scores2(cq_ref[...], relc_cq_ref[0])
    s_ref[...] += (jnp.sum(jnp.exp2(y), axis=1, keepdims=True)
                   + jnp.sum(jnp.exp2(yc), axis=1, keepdims=True))

    col = jax.lax.broadcasted_iota(jnp.int32, (BATCH, BLK), 1)
    hit = col == labels - i * BLK
    lsc_ref[...] += jnp.sum(jnp.where(hit, y, 0.0), axis=1, keepdims=True)

    @pl.when(i == NUM_STEPS - 1)
    def _finish():
        lse = jnp.log(s_ref[...])
        nll = lse - lsc_ref[...] * LN2
        valid = (labels != IGNORE_INDEX).astype(jnp.float32)
        loss = (jnp.sum(nll * valid, keepdims=True)
                / jnp.maximum(jnp.sum(valid, keepdims=True), 1.0))
        out_ref[...] = loss.reshape(1, 1)


def kernel(inputs, roi_label, roi_ious, lut, cq, reliability):
    del roi_ious
    labels = (roi_label.reshape(-1) - 1).astype(jnp.int32).reshape(BATCH, 1)
    relc = reliability * jnp.float32(OIM_SCALAR * LOG2E)
    relc_lut = relc[:NUM_PIDS].reshape(NUM_STEPS, 1, BLK)
    relc_cq = relc[NUM_PIDS:].reshape(NUM_STEPS, 1, CBLK)

    out = pl.pallas_call(
        _oim_kernel,
        grid=(NUM_STEPS,),
        in_specs=[
            pl.BlockSpec((BATCH, NUM_FEATURES), lambda i: (0, 0)),
            pl.BlockSpec((BATCH, 1), lambda i: (0, 0)),
            pl.BlockSpec((BLK, NUM_FEATURES), lambda i: (i, 0)),
            pl.BlockSpec((CBLK, NUM_FEATURES), lambda i: (i, 0)),
            pl.BlockSpec((1, 1, BLK), lambda i: (i, 0, 0)),
            pl.BlockSpec((1, 1, CBLK), lambda i: (i, 0, 0)),
        ],
        out_specs=pl.BlockSpec((1, 1), lambda i: (0, 0)),
        out_shape=jax.ShapeDtypeStruct((1, 1), jnp.float32),
        scratch_shapes=[
            pltpu.VMEM((BATCH, 1), jnp.float32),
            pltpu.VMEM((BATCH, 1), jnp.float32),
        ],
    )(inputs, labels, lut, cq, relc_lut, relc_cq)
    return out[0, 0]


# speedup vs baseline: 1.1118x; 1.1118x over previous
import math

import jax
import jax.numpy as jnp
from jax.experimental import pallas as pl
from jax.experimental.pallas import tpu as pltpu

NUM_FEATURES = 128
NUM_PIDS = 100000
NUM_CQ = 5000
OIM_SCALAR = 30.0
BATCH = 128
CHUNK = 10000
NCH = NUM_PIDS // CHUNK
NBUF = 4
IGNORE_INDEX = 5554
LOG2E = math.log2(math.e)
LN2 = math.log(2.0)


def _oim_kernel(x_ref, lab_ref, relc_lut_ref, relc_cq_ref, lut_ref, cq_ref,
                out_ref, buf, cqbuf, sems, cqsem):
    x = x_ref[...]
    labels = lab_ref[...]

    def lut_copy(k):
        return pltpu.make_async_copy(
            lut_ref.at[pl.ds(k * CHUNK, CHUNK), :],
            buf.at[k % NBUF], sems.at[k % NBUF])

    def scores2(w, relc):
        lg = jax.lax.dot_general(
            x, w, (((1,), (1,)), ((), ())),
            preferred_element_type=jnp.float32,
            precision=jax.lax.Precision.DEFAULT)
        return lg * relc

    lut_copy(0).start()
    cq_cp = pltpu.make_async_copy(cq_ref, cqbuf, cqsem)
    cq_cp.start()
    for k in range(1, NBUF):
        lut_copy(k).start()

    s = jnp.zeros((BATCH, 1), jnp.float32)
    lsc = jnp.zeros((BATCH, 1), jnp.float32)
    col = jax.lax.broadcasted_iota(jnp.int32, (BATCH, CHUNK), 1)
    for k in range(NCH):
        lut_copy(k).wait()
        y = scores2(buf[k % NBUF], relc_lut_ref[k])
        s = s + jnp.sum(jnp.exp2(y), axis=1, keepdims=True)
        hit = col == labels - k * CHUNK
        lsc = lsc + jnp.sum(jnp.where(hit, y, 0.0), axis=1, keepdims=True)
        if k + NBUF < NCH:
            lut_copy(k + NBUF).start()

    cq_cp.wait()
    yc = scores2(cqbuf[...], relc_cq_ref[...])
    s = s + jnp.sum(jnp.exp2(yc), axis=1, keepdims=True)

    lse = jnp.log(s)
    nll = lse - lsc * LN2
    valid = (labels != IGNORE_INDEX).astype(jnp.float32)
    loss = (jnp.sum(nll * valid, keepdims=True)
            / jnp.maximum(jnp.sum(valid, keepdims=True), 1.0))
    out_ref[...] = loss.reshape(1, 1)


def kernel(inputs, roi_label, roi_ious, lut, cq, reliability):
    del roi_ious
    labels = (roi_label.reshape(-1) - 1).astype(jnp.int32).reshape(BATCH, 1)
    relc = reliability * jnp.float32(OIM_SCALAR * LOG2E)
    relc_lut = relc[:NUM_PIDS].reshape(NCH, 1, CHUNK)
    relc_cq = relc[NUM_PIDS:].reshape(1, NUM_CQ)

    vmem = pltpu.MemorySpace.VMEM
    out = pl.pallas_call(
        _oim_kernel,
        in_specs=[
            pl.BlockSpec(memory_space=vmem),
            pl.BlockSpec(memory_space=vmem),
            pl.BlockSpec(memory_space=vmem),
            pl.BlockSpec(memory_space=vmem),
            pl.BlockSpec(memory_space=pltpu.MemorySpace.HBM),
            pl.BlockSpec(memory_space=pltpu.MemorySpace.HBM),
        ],
        out_specs=pl.BlockSpec(memory_space=vmem),
        out_shape=jax.ShapeDtypeStruct((1, 1), jnp.float32),
        scratch_shapes=[
            pltpu.VMEM((NBUF, CHUNK, NUM_FEATURES), jnp.float32),
            pltpu.VMEM((NUM_CQ, NUM_FEATURES), jnp.float32),
            pltpu.SemaphoreType.DMA((NBUF,)),
            pltpu.SemaphoreType.DMA,
        ],
    )(inputs, labels, relc_lut, relc_cq, lut, cq)
    return out[0, 0]
